# Initial kernel scaffold; baseline (speedup 1.0000x reference)
import jax
import jax.numpy as jnp
from jax.experimental import pallas as pl


def kernel(logits, targs):
    raise NotImplementedError("write your pallas kernel here")


# speedup vs baseline: 1.2765x; 1.2765x over previous
import functools

import jax
import jax.numpy as jnp
import numpy as np
from jax.experimental import pallas as pl
from jax.experimental.pallas import tpu as pltpu

NBINS = 15
N_ROWS = 16384
N_COLS = 1000
BLOCK_ROWS = 1024
GRID = N_ROWS // BLOCK_ROWS

_BOUNDS = np.linspace(0.0, 1.0, NBINS + 1).astype(np.float32)
_LOWER = _BOUNDS[:-1].reshape(1, NBINS)
_UPPER = _BOUNDS[1:].reshape(1, NBINS)


def _ece_kernel(logits_ref, targs_ref, bounds_ref, out_ref, acc_ref):
    step = pl.program_id(0)

    @pl.when(step == 0)
    def _init():
        acc_ref[...] = jnp.zeros_like(acc_ref)

    x = logits_ref[...]
    m = jnp.max(x, axis=1, keepdims=True)
    s = jnp.sum(jnp.exp(x - m), axis=1, keepdims=True)
    conf = 1.0 / s

    col = jax.lax.broadcasted_iota(jnp.int32, x.shape, 1)
    pred = jnp.min(jnp.where(x == m, col, N_COLS), axis=1)
    targ = targs_ref[0, 0, :]
    acc = (pred == targ).astype(jnp.float32)[:, None]

    lower = bounds_ref[0:1, :]
    upper = bounds_ref[1:2, :]
    in_bin = ((conf > lower) & (conf <= upper)).astype(jnp.float32)
    acc_ref[0, :] += jnp.sum(in_bin, axis=0)
    acc_ref[1, :] += jnp.sum(in_bin * conf, axis=0)
    acc_ref[2, :] += jnp.sum(in_bin * acc, axis=0)

    @pl.when(step == GRID - 1)
    def _fini():
        cnt = acc_ref[0, :]
        conf_sum = acc_ref[1, :]
        acc_sum = acc_ref[2, :]
        safe = jnp.maximum(cnt, 1.0)
        prop = cnt * (1.0 / N_ROWS)
        gap = jnp.abs(conf_sum / safe - acc_sum / safe) * prop
        ece = jnp.sum(jnp.where(cnt > 0.0, gap, 0.0))
        out_ref[...] = ece.reshape(1, 1)


@jax.jit
def kernel(logits, targs):
    targs3 = targs.reshape(GRID, 1, BLOCK_ROWS)
    bounds = jnp.asarray(np.concatenate([_LOWER, _UPPER], axis=0))
    out = pl.pallas_call(
        _ece_kernel,
        grid=(GRID,),
        in_specs=[
            pl.BlockSpec((BLOCK_ROWS, N_COLS), lambda i: (i, 0)),
            pl.BlockSpec((1, 1, BLOCK_ROWS), lambda i: (i, 0, 0)),
            pl.BlockSpec((2, NBINS), lambda i: (0, 0)),
        ],
        out_specs=pl.BlockSpec((1, 1), lambda i: (0, 0)),
        out_shape=jax.ShapeDtypeStruct((1, 1), jnp.float32),
        scratch_shapes=[pltpu.VMEM((3, NBINS), jnp.float32)],
    )(logits, targs3, bounds)
    return out.reshape(1)
